# baseline (speedup 1.0000x reference)
import jax
import jax.numpy as jnp
from jax import lax
from jax.experimental import pallas as pl
from jax.experimental.pallas import tpu as pltpu

N = 576
D = 768
K = 4
H = N // K


def _body(x_ref, tok_ref, pos_hbm, out_hbm, pos_v, out_v, sem_pos, sem_out):
    pos_cp = pltpu.make_async_copy(pos_hbm.at[pl.ds(0, N), :], pos_v, sem_pos)
    pos_cp.start()
    xv = x_ref[...]
    tok_b = tok_ref[...].astype(jnp.bfloat16)
    pos_ref = pos_v
    cps = []
    for k in range(K):
        xs = xv[k * H:(k + 1) * H]
        iota = lax.broadcasted_iota(jnp.int32, (N, H), 0)
        oh_t = (iota == xs[None, :]).astype(jnp.bfloat16)
        y = lax.dot_general(
            oh_t, tok_b, (((0,), (0,)), ((), ())),
            preferred_element_type=jnp.float32,
        )
        if k == 0:
            pos_cp.wait()
        out_v[k] = y + pos_ref[k * H:(k + 1) * H, :]
        cp = pltpu.make_async_copy(
            out_v.at[k], out_hbm.at[pl.ds(k * H, H), :], sem_out
        )
        cp.start()
        cps.append(cp)
    for cp in cps:
        cp.wait()


def kernel(x, token_table, pos_table):
    return pl.pallas_call(
        _body,
        out_shape=jax.ShapeDtypeStruct((N, D), jnp.float32),
        grid=(1,),
        in_specs=[
            pl.BlockSpec((N,), lambda i: (0,)),
            pl.BlockSpec((N, D), lambda i: (0, 0)),
            pl.BlockSpec(memory_space=pl.ANY),
        ],
        out_specs=pl.BlockSpec(memory_space=pl.ANY),
        scratch_shapes=[
            pltpu.VMEM((N, D), jnp.float32),
            pltpu.VMEM((K, H, D), jnp.float32),
            pltpu.SemaphoreType.DMA,
            pltpu.SemaphoreType.DMA,
        ],
    )(x, token_table, pos_table)


# speedup vs baseline: 1.2371x; 1.2371x over previous
import jax
import jax.numpy as jnp
from jax import lax
from jax.experimental import pallas as pl
from jax.experimental.pallas import tpu as pltpu

N = 576
D = 768
K = 4
H = N // K


def _body(x_ref, tok_ref, pos_ref, out_hbm, out_v, sem):
    xv = x_ref[...]
    tok_b = tok_ref[...].astype(jnp.bfloat16)
    cps = []
    for k in range(K):
        xs = xv[k * H:(k + 1) * H]
        iota = lax.broadcasted_iota(jnp.int32, (N, H), 0)
        oh_t = (iota == xs[None, :]).astype(jnp.bfloat16)
        y = lax.dot_general(
            oh_t, tok_b, (((0,), (0,)), ((), ())),
            preferred_element_type=jnp.float32,
        )
        out_v[k] = y + pos_ref[k * H:(k + 1) * H, :]
        cp = pltpu.make_async_copy(
            out_v.at[k], out_hbm.at[pl.ds(k * H, H), :], sem
        )
        cp.start()
        cps.append(cp)
    for cp in cps:
        cp.wait()


def kernel(x, token_table, pos_table):
    return pl.pallas_call(
        _body,
        out_shape=jax.ShapeDtypeStruct((N, D), jnp.float32),
        grid=(1,),
        in_specs=[
            pl.BlockSpec((N,), lambda i: (0,)),
            pl.BlockSpec((N, D), lambda i: (0, 0)),
            pl.BlockSpec((N, D), lambda i: (0, 0)),
        ],
        out_specs=pl.BlockSpec(memory_space=pl.ANY),
        scratch_shapes=[
            pltpu.VMEM((K, H, D), jnp.float32),
            pltpu.SemaphoreType.DMA,
        ],
    )(x, token_table, pos_table)
